# Initial kernel scaffold; baseline (speedup 1.0000x reference)
import jax
import jax.numpy as jnp
from jax.experimental import pallas as pl


def kernel(inputs, pos_table):
    raise NotImplementedError("write your pallas kernel here")


# speedup vs baseline: 1.7218x; 1.7218x over previous
import jax
import jax.numpy as jnp
from jax.experimental import pallas as pl

_CHUNK = 256


def _add_kernel(x_ref, p_ref, o_ref):
    o_ref[...] = x_ref[...] + p_ref[...][None, :, :]


def kernel(inputs, pos_table):
    b, s, d = inputs.shape
    chunk = min(_CHUNK, s)
    return pl.pallas_call(
        _add_kernel,
        grid=(s // chunk,),
        in_specs=[
            pl.BlockSpec((b, chunk, d), lambda i: (0, i, 0)),
            pl.BlockSpec((chunk, d), lambda i: (i, 0)),
        ],
        out_specs=pl.BlockSpec((b, chunk, d), lambda i: (0, i, 0)),
        out_shape=jax.ShapeDtypeStruct((b, s, d), inputs.dtype),
    )(inputs, pos_table)


# speedup vs baseline: 1.7292x; 1.0043x over previous
import jax
import jax.numpy as jnp
from jax.experimental import pallas as pl

_CHUNK = 512


def _add_kernel(x_ref, p_ref, o_ref):
    o_ref[...] = x_ref[...] + p_ref[...][None, :, :]


def kernel(inputs, pos_table):
    b, s, d = inputs.shape
    chunk = min(_CHUNK, s)
    return pl.pallas_call(
        _add_kernel,
        grid=(s // chunk,),
        in_specs=[
            pl.BlockSpec((b, chunk, d), lambda i: (0, i, 0)),
            pl.BlockSpec((chunk, d), lambda i: (i, 0)),
        ],
        out_specs=pl.BlockSpec((b, chunk, d), lambda i: (0, i, 0)),
        out_shape=jax.ShapeDtypeStruct((b, s, d), inputs.dtype),
    )(inputs, pos_table)
